# baseline (speedup 1.0000x reference)
import functools

import jax
import jax.numpy as jnp
from jax.experimental import pallas as pl
from jax.experimental.pallas import tpu as pltpu

F = 128
SHARED = 2048
ALPHA = 0.1
BM = 512


def _leaky(x):
    return jnp.where(x > 0.0, x, ALPHA * x)


def _mm_kernel(x_ref, w_ref, o_ref):
    o_ref[...] = jnp.dot(
        x_ref[...], w_ref[0], preferred_element_type=jnp.float32
    ).astype(jnp.bfloat16)


def _mm2(x, w2):
    M2 = x.shape[0]
    hb = (M2 // 2) // BM
    return pl.pallas_call(
        _mm_kernel,
        grid=(M2 // BM,),
        in_specs=[
            pl.BlockSpec((BM, F), lambda m: (m, 0)),
            pl.BlockSpec((1, F, F), lambda m: (m // hb, 0, 0)),
        ],
        out_specs=pl.BlockSpec((BM, F), lambda m: (m, 0)),
        out_shape=jax.ShapeDtypeStruct((M2, F), jnp.bfloat16),
    )(x, w2)


def _gcn_post_kernel(adj_ref, y_ref, b_ref, pw_ref, o_ref):
    acc = jnp.dot(adj_ref[...], y_ref[0], preferred_element_type=jnp.float32)
    z = _leaky(acc + b_ref[0])
    o_ref[...] = jnp.dot(
        z.astype(jnp.bfloat16), pw_ref[0], preferred_element_type=jnp.float32
    ).astype(jnp.bfloat16)


def _gcn_post(adj, y2, b2, pw2):
    M2, K = adj.shape
    hb = (M2 // 2) // BM
    return pl.pallas_call(
        _gcn_post_kernel,
        grid=(M2 // BM,),
        in_specs=[
            pl.BlockSpec((BM, K), lambda m: (m, 0)),
            pl.BlockSpec((1, K, F), lambda m: (m // hb, 0, 0)),
            pl.BlockSpec((1, 1, F), lambda m: (m // hb, 0, 0)),
            pl.BlockSpec((1, F, F), lambda m: (m // hb, 0, 0)),
        ],
        out_specs=pl.BlockSpec((BM, F), lambda m: (m, 0)),
        out_shape=jax.ShapeDtypeStruct((M2, F), jnp.bfloat16),
        compiler_params=pltpu.CompilerParams(
            dimension_semantics=("arbitrary",)
        ),
    )(adj, y2, b2, pw2)


def _gcn_union_kernel(adj_ref, y_ref, b_ref, wa_ref, x_ref, wb_ref, ub_ref,
                      o_ref, *, relu):
    acc = jnp.dot(adj_ref[...], y_ref[0], preferred_element_type=jnp.float32)
    z = _leaky(acc + b_ref[0])
    o = (
        jnp.dot(z.astype(jnp.bfloat16), wa_ref[0],
                preferred_element_type=jnp.float32)
        + jnp.dot(x_ref[...], wb_ref[0], preferred_element_type=jnp.float32)
        + ub_ref[0]
    )
    if relu:
        o = jnp.maximum(o, 0.0)
    o_ref[...] = o


def _gcn_union(adj, y2, b2, wa2, x, wb2, ub2, relu):
    M2, K = adj.shape
    hb = (M2 // 2) // BM
    return pl.pallas_call(
        functools.partial(_gcn_union_kernel, relu=relu),
        grid=(M2 // BM,),
        in_specs=[
            pl.BlockSpec((BM, K), lambda m: (m, 0)),
            pl.BlockSpec((1, K, F), lambda m: (m // hb, 0, 0)),
            pl.BlockSpec((1, 1, F), lambda m: (m // hb, 0, 0)),
            pl.BlockSpec((1, F, F), lambda m: (m // hb, 0, 0)),
            pl.BlockSpec((BM, F), lambda m: (m, 0)),
            pl.BlockSpec((1, F, F), lambda m: (m // hb, 0, 0)),
            pl.BlockSpec((1, 1, F), lambda m: (m // hb, 0, 0)),
        ],
        out_specs=pl.BlockSpec((BM, F), lambda m: (m, 0)),
        out_shape=jax.ShapeDtypeStruct((M2, F), jnp.float32),
        compiler_params=pltpu.CompilerParams(
            dimension_semantics=("arbitrary",)
        ),
    )(adj, y2, b2, wa2, x, wb2, ub2)


def kernel(source_UV, source_VU, target_UV, target_VU, src_user_emb,
           src_item_emb, tgt_user_emb, tgt_item_emb, src_dgcn_W, src_dgcn_b,
           src_union_W, src_union_b, src_last_W, src_last_b, src_last_union_W,
           src_last_union_b, tgt_dgcn_W, tgt_dgcn_b, tgt_union_W, tgt_union_b,
           tgt_last_W, tgt_last_b, tgt_last_union_W, tgt_last_union_b):
    bf16 = jnp.bfloat16
    NU, NI = source_UV.shape

    def pw(s, t):
        return jnp.stack([s, t]).astype(bf16)

    def pb(s, t):
        return jnp.stack([s, t]).reshape(2, 1, F)

    UVc = jnp.concatenate([source_UV, target_UV], 0).astype(bf16)
    VUc = jnp.concatenate([source_VU, target_VU], 0).astype(bf16)
    u0c = jnp.concatenate([src_user_emb, tgt_user_emb], 0).astype(bf16)
    v0c = jnp.concatenate([src_item_emb, tgt_item_emb], 0).astype(bf16)

    yU = _mm2(u0c, pw(src_dgcn_W[0], tgt_dgcn_W[0]))
    yV = _mm2(v0c, pw(src_dgcn_W[1], tgt_dgcn_W[1]))
    pU = _gcn_post(VUc, yU.reshape(2, NU, F),
                   pb(src_dgcn_b[0], tgt_dgcn_b[0]),
                   pw(src_dgcn_W[2], tgt_dgcn_W[2]))
    pV = _gcn_post(UVc, yV.reshape(2, NI, F),
                   pb(src_dgcn_b[1], tgt_dgcn_b[1]),
                   pw(src_dgcn_W[3], tgt_dgcn_W[3]))
    u1 = _gcn_union(UVc, pU.reshape(2, NI, F),
                    pb(src_dgcn_b[2], tgt_dgcn_b[2]),
                    pw(src_union_W[0][:F], tgt_union_W[0][:F]), u0c,
                    pw(src_union_W[0][F:], tgt_union_W[0][F:]),
                    pb(src_union_b[0], tgt_union_b[0]), relu=True)
    v1 = _gcn_union(VUc, pV.reshape(2, NU, F),
                    pb(src_dgcn_b[3], tgt_dgcn_b[3]),
                    pw(src_union_W[1][:F], tgt_union_W[1][:F]), v0c,
                    pw(src_union_W[1][F:], tgt_union_W[1][F:]),
                    pb(src_union_b[1], tgt_union_b[1]), relu=True)
    u1b = u1.astype(bf16)
    v1b = v1.astype(bf16)

    yU2 = _mm2(u1b, pw(src_last_W[0], tgt_last_W[0]))
    yV2 = _mm2(v1b, pw(src_last_W[1], tgt_last_W[1]))
    pU2 = _gcn_post(VUc, yU2.reshape(2, NU, F),
                    pb(src_last_b[0], tgt_last_b[0]),
                    pw(src_last_W[2], tgt_last_W[2]))
    pV2 = _gcn_post(UVc, yV2.reshape(2, NI, F),
                    pb(src_last_b[1], tgt_last_b[1]),
                    pw(src_last_W[4], tgt_last_W[4]))
    u2 = _gcn_union(UVc, pU2.reshape(2, NI, F),
                    pb(src_last_b[2], tgt_last_b[2]),
                    pw(src_last_union_W[0][:F], tgt_last_union_W[0][:F]), u1b,
                    pw(src_last_union_W[0][F:], tgt_last_union_W[0][F:]),
                    pb(src_last_union_b[0], tgt_last_union_b[0]), relu=False)
    v2 = _gcn_union(VUc, pV2.reshape(2, NU, F),
                    pb(src_last_b[4], tgt_last_b[4]),
                    pw(src_last_union_W[2][:F], tgt_last_union_W[2][:F]), v1b,
                    pw(src_last_union_W[2][F:], tgt_last_union_W[2][F:]),
                    pb(src_last_union_b[2], tgt_last_union_b[2]), relu=False)

    s_user = jnp.concatenate([u1[:NU], u2[:NU]], 1)
    t_user = jnp.concatenate([u1[NU:], u2[NU:]], 1)
    s_item = jnp.concatenate([v1[:NI], v2[:NI]], 1)
    t_item = jnp.concatenate([v1[NI:], v2[NI:]], 1)
    src_concat = jnp.concatenate([t_user[:SHARED], s_user[SHARED:]], 0)
    tgt_concat = jnp.concatenate([s_user[:SHARED], t_user[SHARED:]], 0)
    return src_concat, s_item, tgt_concat, t_item


# speedup vs baseline: 1.2282x; 1.2282x over previous
import functools

import jax
import jax.numpy as jnp
from jax.experimental import pallas as pl
from jax.experimental.pallas import tpu as pltpu

F = 128
SHARED = 2048
ALPHA = 0.1
BM = 512


def _leaky(x):
    return jnp.where(x > 0.0, x, ALPHA * x)


def _mm_kernel(x_ref, w_ref, o_ref):
    o_ref[...] = jnp.dot(
        x_ref[...], w_ref[...], preferred_element_type=jnp.float32
    ).astype(jnp.bfloat16)


def _mm(x, w):
    M = x.shape[0]
    return pl.pallas_call(
        _mm_kernel,
        grid=(M // BM,),
        in_specs=[
            pl.BlockSpec((BM, F), lambda m: (m, 0)),
            pl.BlockSpec((F, F), lambda m: (0, 0)),
        ],
        out_specs=pl.BlockSpec((BM, F), lambda m: (m, 0)),
        out_shape=jax.ShapeDtypeStruct((M, F), jnp.bfloat16),
    )(x, w)


def _gcn_post_kernel(adj_ref, y_ref, b_ref, pw_ref, o_ref):
    acc = jnp.dot(adj_ref[...], y_ref[...], preferred_element_type=jnp.float32)
    z = _leaky(acc + b_ref[...])
    o_ref[...] = jnp.dot(
        z.astype(jnp.bfloat16), pw_ref[...], preferred_element_type=jnp.float32
    ).astype(jnp.bfloat16)


def _gcn_post(adj, y, b, pwt):
    M, K = adj.shape
    return pl.pallas_call(
        _gcn_post_kernel,
        grid=(M // BM,),
        in_specs=[
            pl.BlockSpec((BM, K), lambda m: (m, 0)),
            pl.BlockSpec((K, F), lambda m: (0, 0)),
            pl.BlockSpec((1, F), lambda m: (0, 0)),
            pl.BlockSpec((F, F), lambda m: (0, 0)),
        ],
        out_specs=pl.BlockSpec((BM, F), lambda m: (m, 0)),
        out_shape=jax.ShapeDtypeStruct((M, F), jnp.bfloat16),
        compiler_params=pltpu.CompilerParams(
            dimension_semantics=("arbitrary",)
        ),
    )(adj, y, b.reshape(1, F), pwt)


def _gcn_union_kernel(adj_ref, y_ref, b_ref, wa_ref, x_ref, wb_ref, ub_ref,
                      o_ref, *, relu):
    acc = jnp.dot(adj_ref[...], y_ref[...], preferred_element_type=jnp.float32)
    z = _leaky(acc + b_ref[...])
    o = (
        jnp.dot(z.astype(jnp.bfloat16), wa_ref[...],
                preferred_element_type=jnp.float32)
        + jnp.dot(x_ref[...], wb_ref[...], preferred_element_type=jnp.float32)
        + ub_ref[...]
    )
    if relu:
        o = jnp.maximum(o, 0.0)
    o_ref[...] = o


def _gcn_union(adj, y, b, wa, x, wb, ub, relu):
    M, K = adj.shape
    return pl.pallas_call(
        functools.partial(_gcn_union_kernel, relu=relu),
        grid=(M // BM,),
        in_specs=[
            pl.BlockSpec((BM, K), lambda m: (m, 0)),
            pl.BlockSpec((K, F), lambda m: (0, 0)),
            pl.BlockSpec((1, F), lambda m: (0, 0)),
            pl.BlockSpec((F, F), lambda m: (0, 0)),
            pl.BlockSpec((BM, F), lambda m: (m, 0)),
            pl.BlockSpec((F, F), lambda m: (0, 0)),
            pl.BlockSpec((1, F), lambda m: (0, 0)),
        ],
        out_specs=pl.BlockSpec((BM, F), lambda m: (m, 0)),
        out_shape=jax.ShapeDtypeStruct((M, F), jnp.float32),
        compiler_params=pltpu.CompilerParams(
            dimension_semantics=("arbitrary",)
        ),
    )(adj, y, b.reshape(1, F), wa, x, wb, ub.reshape(1, F))


def _vbge(UV, VU, u0, v0, dW, db, uW, ub, lW, lb, luW, lub):
    bf16 = jnp.bfloat16
    u0b = u0.astype(bf16)
    v0b = v0.astype(bf16)
    dWb = dW.astype(bf16)
    uWb = uW.astype(bf16)
    lWb = lW.astype(bf16)
    luWb = luW.astype(bf16)

    yU = _mm(u0b, dWb[0])
    yV = _mm(v0b, dWb[1])
    pU = _gcn_post(VU, yU, db[0], dWb[2])
    pV = _gcn_post(UV, yV, db[1], dWb[3])
    u1 = _gcn_union(UV, pU, db[2], uWb[0][:F], u0b, uWb[0][F:], ub[0],
                    relu=True)
    v1 = _gcn_union(VU, pV, db[3], uWb[1][:F], v0b, uWb[1][F:], ub[1],
                    relu=True)
    u1b = u1.astype(bf16)
    v1b = v1.astype(bf16)

    yU2 = _mm(u1b, lWb[0])
    yV2 = _mm(v1b, lWb[1])
    pU2 = _gcn_post(VU, yU2, lb[0], lWb[2])
    pV2 = _gcn_post(UV, yV2, lb[1], lWb[4])
    u2 = _gcn_union(UV, pU2, lb[2], luWb[0][:F], u1b, luWb[0][F:], lub[0],
                    relu=False)
    v2 = _gcn_union(VU, pV2, lb[4], luWb[2][:F], v1b, luWb[2][F:], lub[2],
                    relu=False)
    return (jnp.concatenate([u1, u2], 1), jnp.concatenate([v1, v2], 1))


def kernel(source_UV, source_VU, target_UV, target_VU, src_user_emb,
           src_item_emb, tgt_user_emb, tgt_item_emb, src_dgcn_W, src_dgcn_b,
           src_union_W, src_union_b, src_last_W, src_last_b, src_last_union_W,
           src_last_union_b, tgt_dgcn_W, tgt_dgcn_b, tgt_union_W, tgt_union_b,
           tgt_last_W, tgt_last_b, tgt_last_union_W, tgt_last_union_b):
    bf16 = jnp.bfloat16
    sUV = source_UV.astype(bf16)
    sVU = source_VU.astype(bf16)
    tUV = target_UV.astype(bf16)
    tVU = target_VU.astype(bf16)

    s_user, s_item = _vbge(sUV, sVU, src_user_emb, src_item_emb, src_dgcn_W,
                           src_dgcn_b, src_union_W, src_union_b, src_last_W,
                           src_last_b, src_last_union_W, src_last_union_b)
    t_user, t_item = _vbge(tUV, tVU, tgt_user_emb, tgt_item_emb, tgt_dgcn_W,
                           tgt_dgcn_b, tgt_union_W, tgt_union_b, tgt_last_W,
                           tgt_last_b, tgt_last_union_W, tgt_last_union_b)

    src_concat = jnp.concatenate([t_user[:SHARED], s_user[SHARED:]], 0)
    tgt_concat = jnp.concatenate([s_user[:SHARED], t_user[SHARED:]], 0)
    return src_concat, s_item, tgt_concat, t_item


# speedup vs baseline: 1.4121x; 1.1498x over previous
import functools

import jax
import jax.numpy as jnp
from jax.experimental import pallas as pl
from jax.experimental.pallas import tpu as pltpu

F = 128
SHARED = 2048
ALPHA = 0.1
BM = 512
BMF = 256

_ARB = pltpu.CompilerParams(dimension_semantics=("arbitrary",))


def _leaky(x):
    return jnp.where(x > 0.0, x, ALPHA * x)


def _bdot(a, b):
    return jnp.dot(a, b, preferred_element_type=jnp.float32)


def _mm_kernel(x_ref, w_ref, o_ref):
    o_ref[...] = _bdot(
        x_ref[...].astype(jnp.bfloat16), w_ref[...]
    ).astype(jnp.bfloat16)


def _mm(x, w):
    M = x.shape[0]
    return pl.pallas_call(
        _mm_kernel,
        grid=(M // BM,),
        in_specs=[
            pl.BlockSpec((BM, F), lambda m: (m, 0)),
            pl.BlockSpec((F, F), lambda m: (0, 0)),
        ],
        out_specs=pl.BlockSpec((BM, F), lambda m: (m, 0)),
        out_shape=jax.ShapeDtypeStruct((M, F), jnp.bfloat16),
    )(x, w)


def _p1_kernel(adj_ref, y_ref, b_ref, pw_ref, o_ref, adjb_ref):
    ab = adj_ref[...].astype(jnp.bfloat16)
    adjb_ref[...] = ab
    z = _leaky(_bdot(ab, y_ref[...]) + b_ref[...])
    o_ref[...] = _bdot(z.astype(jnp.bfloat16), pw_ref[...]).astype(jnp.bfloat16)


def _p1(adj, y, b, pwt):
    M, K = adj.shape
    return pl.pallas_call(
        _p1_kernel,
        grid=(M // BMF,),
        in_specs=[
            pl.BlockSpec((BMF, K), lambda m: (m, 0)),
            pl.BlockSpec((K, F), lambda m: (0, 0)),
            pl.BlockSpec((1, F), lambda m: (0, 0)),
            pl.BlockSpec((F, F), lambda m: (0, 0)),
        ],
        out_specs=[
            pl.BlockSpec((BMF, F), lambda m: (m, 0)),
            pl.BlockSpec((BMF, K), lambda m: (m, 0)),
        ],
        out_shape=[
            jax.ShapeDtypeStruct((M, F), jnp.bfloat16),
            jax.ShapeDtypeStruct((M, K), jnp.bfloat16),
        ],
        compiler_params=_ARB,
    )(adj, y, b.reshape(1, F), pwt)


def _vu_kernel(adj_ref, y1_ref, b1_ref, pw_ref, y2_ref, b2_ref, wa_ref, x_ref,
               wb_ref, ub_ref, nw_ref, p_ref, o_ref, ny_ref, *, relu):
    ab = adj_ref[...].astype(jnp.bfloat16)
    z1 = _leaky(_bdot(ab, y1_ref[...]) + b1_ref[...])
    p_ref[...] = _bdot(z1.astype(jnp.bfloat16), pw_ref[...]).astype(jnp.bfloat16)
    z2 = _leaky(_bdot(ab, y2_ref[...]) + b2_ref[...])
    o = (_bdot(z2.astype(jnp.bfloat16), wa_ref[...])
         + _bdot(x_ref[...].astype(jnp.bfloat16), wb_ref[...])
         + ub_ref[...])
    if relu:
        o = jnp.maximum(o, 0.0)
    o_ref[...] = o
    ny_ref[...] = _bdot(o.astype(jnp.bfloat16), nw_ref[...]).astype(jnp.bfloat16)


def _vu_pass(adj, y1, b1, pwt, y2, b2, wa, x, wb, ub, nw, relu):
    M, K = adj.shape
    return pl.pallas_call(
        functools.partial(_vu_kernel, relu=relu),
        grid=(M // BM,),
        in_specs=[
            pl.BlockSpec((BM, K), lambda m: (m, 0)),
            pl.BlockSpec((K, F), lambda m: (0, 0)),
            pl.BlockSpec((1, F), lambda m: (0, 0)),
            pl.BlockSpec((F, F), lambda m: (0, 0)),
            pl.BlockSpec((K, F), lambda m: (0, 0)),
            pl.BlockSpec((1, F), lambda m: (0, 0)),
            pl.BlockSpec((F, F), lambda m: (0, 0)),
            pl.BlockSpec((BM, F), lambda m: (m, 0)),
            pl.BlockSpec((F, F), lambda m: (0, 0)),
            pl.BlockSpec((1, F), lambda m: (0, 0)),
            pl.BlockSpec((F, F), lambda m: (0, 0)),
        ],
        out_specs=[
            pl.BlockSpec((BM, F), lambda m: (m, 0)),
            pl.BlockSpec((BM, F), lambda m: (m, 0)),
            pl.BlockSpec((BM, F), lambda m: (m, 0)),
        ],
        out_shape=[
            jax.ShapeDtypeStruct((M, F), jnp.bfloat16),
            jax.ShapeDtypeStruct((M, F), jnp.float32),
            jax.ShapeDtypeStruct((M, F), jnp.bfloat16),
        ],
        compiler_params=_ARB,
    )(adj, y1, b1.reshape(1, F), pwt, y2, b2.reshape(1, F), wa, x, wb,
      ub.reshape(1, F), nw)


def _uv2_kernel(adjb_ref, y1_ref, b1_ref, wa_ref, x_ref, wb_ref, ub_ref,
                nw_ref, y2_ref, b2_ref, pw_ref, u_ref, ny_ref, p_ref):
    ab = adjb_ref[...]
    z1 = _leaky(_bdot(ab, y1_ref[...]) + b1_ref[...])
    u = jnp.maximum(
        _bdot(z1.astype(jnp.bfloat16), wa_ref[...])
        + _bdot(x_ref[...].astype(jnp.bfloat16), wb_ref[...])
        + ub_ref[...], 0.0)
    u_ref[...] = u
    ny_ref[...] = _bdot(u.astype(jnp.bfloat16), nw_ref[...]).astype(jnp.bfloat16)
    z2 = _leaky(_bdot(ab, y2_ref[...]) + b2_ref[...])
    p_ref[...] = _bdot(z2.astype(jnp.bfloat16), pw_ref[...]).astype(jnp.bfloat16)


def _uv2_pass(adjb, y1, b1, wa, x, wb, ub, nw, y2, b2, pwt):
    M, K = adjb.shape
    return pl.pallas_call(
        _uv2_kernel,
        grid=(M // BM,),
        in_specs=[
            pl.BlockSpec((BM, K), lambda m: (m, 0)),
            pl.BlockSpec((K, F), lambda m: (0, 0)),
            pl.BlockSpec((1, F), lambda m: (0, 0)),
            pl.BlockSpec((F, F), lambda m: (0, 0)),
            pl.BlockSpec((BM, F), lambda m: (m, 0)),
            pl.BlockSpec((F, F), lambda m: (0, 0)),
            pl.BlockSpec((1, F), lambda m: (0, 0)),
            pl.BlockSpec((F, F), lambda m: (0, 0)),
            pl.BlockSpec((K, F), lambda m: (0, 0)),
            pl.BlockSpec((1, F), lambda m: (0, 0)),
            pl.BlockSpec((F, F), lambda m: (0, 0)),
        ],
        out_specs=[
            pl.BlockSpec((BM, F), lambda m: (m, 0)),
            pl.BlockSpec((BM, F), lambda m: (m, 0)),
            pl.BlockSpec((BM, F), lambda m: (m, 0)),
        ],
        out_shape=[
            jax.ShapeDtypeStruct((M, F), jnp.float32),
            jax.ShapeDtypeStruct((M, F), jnp.bfloat16),
            jax.ShapeDtypeStruct((M, F), jnp.bfloat16),
        ],
        compiler_params=_ARB,
    )(adjb, y1, b1.reshape(1, F), wa, x, wb, ub.reshape(1, F), nw, y2,
      b2.reshape(1, F), pwt)


def _uv3_kernel(adjb_ref, y_ref, b_ref, wa_ref, x_ref, wb_ref, ub_ref, o_ref):
    z = _leaky(_bdot(adjb_ref[...], y_ref[...]) + b_ref[...])
    o_ref[...] = (_bdot(z.astype(jnp.bfloat16), wa_ref[...])
                  + _bdot(x_ref[...].astype(jnp.bfloat16), wb_ref[...])
                  + ub_ref[...])


def _uv3_pass(adjb, y, b, wa, x, wb, ub):
    M, K = adjb.shape
    return pl.pallas_call(
        _uv3_kernel,
        grid=(M // BM,),
        in_specs=[
            pl.BlockSpec((BM, K), lambda m: (m, 0)),
            pl.BlockSpec((K, F), lambda m: (0, 0)),
            pl.BlockSpec((1, F), lambda m: (0, 0)),
            pl.BlockSpec((F, F), lambda m: (0, 0)),
            pl.BlockSpec((BM, F), lambda m: (m, 0)),
            pl.BlockSpec((F, F), lambda m: (0, 0)),
            pl.BlockSpec((1, F), lambda m: (0, 0)),
        ],
        out_specs=pl.BlockSpec((BM, F), lambda m: (m, 0)),
        out_shape=jax.ShapeDtypeStruct((M, F), jnp.float32),
        compiler_params=_ARB,
    )(adjb, y, b.reshape(1, F), wa, x, wb, ub.reshape(1, F))


def _vbge(UV, VU, u0, v0, dW, db, uW, ub, lW, lb, luW, lub):
    bf16 = jnp.bfloat16
    dWb = dW.astype(bf16)
    uWb = uW.astype(bf16)
    lWb = lW.astype(bf16)
    luWb = luW.astype(bf16)

    yU = _mm(u0, dWb[0])
    yV = _mm(v0, dWb[1])
    pV, UVb = _p1(UV, yV, db[1], dWb[3])
    pU, v1, yV2 = _vu_pass(VU, yU, db[0], dWb[2], pV, db[3], uWb[1][:F], v0,
                           uWb[1][F:], ub[1], lWb[1], relu=True)
    u1, yU2, pV2 = _uv2_pass(UVb, pU, db[2], uWb[0][:F], u0, uWb[0][F:],
                             ub[0], lWb[0], yV2, lb[1], lWb[4])
    pU2, v2, _ = _vu_pass(VU, yU2, lb[0], lWb[2], pV2, lb[4], luWb[2][:F], v1,
                          luWb[2][F:], lub[2], luWb[2][:F], relu=False)
    u2 = _uv3_pass(UVb, pU2, lb[2], luWb[0][:F], u1, luWb[0][F:], lub[0])
    return (jnp.concatenate([u1, u2], 1), jnp.concatenate([v1, v2], 1))


def kernel(source_UV, source_VU, target_UV, target_VU, src_user_emb,
           src_item_emb, tgt_user_emb, tgt_item_emb, src_dgcn_W, src_dgcn_b,
           src_union_W, src_union_b, src_last_W, src_last_b, src_last_union_W,
           src_last_union_b, tgt_dgcn_W, tgt_dgcn_b, tgt_union_W, tgt_union_b,
           tgt_last_W, tgt_last_b, tgt_last_union_W, tgt_last_union_b):
    s_user, s_item = _vbge(source_UV, source_VU, src_user_emb, src_item_emb,
                           src_dgcn_W, src_dgcn_b, src_union_W, src_union_b,
                           src_last_W, src_last_b, src_last_union_W,
                           src_last_union_b)
    t_user, t_item = _vbge(target_UV, target_VU, tgt_user_emb, tgt_item_emb,
                           tgt_dgcn_W, tgt_dgcn_b, tgt_union_W, tgt_union_b,
                           tgt_last_W, tgt_last_b, tgt_last_union_W,
                           tgt_last_union_b)

    src_concat = jnp.concatenate([t_user[:SHARED], s_user[SHARED:]], 0)
    tgt_concat = jnp.concatenate([s_user[:SHARED], t_user[SHARED:]], 0)
    return src_concat, s_item, tgt_concat, t_item


# speedup vs baseline: 1.6711x; 1.1834x over previous
import jax
import jax.numpy as jnp
from jax.experimental import pallas as pl
from jax.experimental.pallas import tpu as pltpu

F = 128
SHARED = 2048
ALPHA = 0.1
BMU = 256
BMV = 512


def _leaky(x):
    return jnp.where(x > 0.0, x, ALPHA * x)


def _bdot(a, b):
    return jnp.dot(a, b, preferred_element_type=jnp.float32)


def _mega_kernel(uv_ref, vu_ref, u0_ref, v0_ref, w_ref, b_ref,
                 u1_ref, u2_ref, v1_ref, v2_ref,
                 syu_ref, syv_ref, spv_ref, spu_ref, su1_ref, sv1_ref,
                 *, nu, nv):
    bf16 = jnp.bfloat16
    m = pl.program_id(0)
    p1s = 1
    p2s = p1s + nu
    p3s = p2s + nv
    p4s = p3s + nu
    p5s = p4s + nv

    @pl.when(m == 0)
    def _():
        syu_ref[...] = _bdot(u0_ref[...].astype(bf16),
                             w_ref[0]).astype(bf16)
        syv_ref[...] = _bdot(v0_ref[...].astype(bf16),
                             w_ref[1]).astype(bf16)

    @pl.when((m >= p1s) & (m < p2s))
    def _():
        i = m - p1s
        ab = uv_ref[...].astype(bf16)
        z = _leaky(_bdot(ab, syv_ref[...]) + b_ref[1])
        spv_ref[pl.ds(i * BMU, BMU), :] = _bdot(
            z.astype(bf16), w_ref[3]).astype(bf16)

    @pl.when((m >= p2s) & (m < p3s))
    def _():
        i = m - p2s
        ab = vu_ref[...].astype(bf16)
        z1 = _leaky(_bdot(ab, syu_ref[...]) + b_ref[0])
        spu_ref[pl.ds(i * BMV, BMV), :] = _bdot(
            z1.astype(bf16), w_ref[2]).astype(bf16)
        z2 = _leaky(_bdot(ab, spv_ref[...]) + b_ref[3])
        v0b = v0_ref[pl.ds(i * BMV, BMV), :].astype(bf16)
        v1 = jnp.maximum(
            _bdot(z2.astype(bf16), w_ref[6]) + _bdot(v0b, w_ref[7])
            + b_ref[5], 0.0)
        v1_ref[...] = v1
        v1b = v1.astype(bf16)
        sv1_ref[pl.ds(i * BMV, BMV), :] = v1b
        syv_ref[pl.ds(i * BMV, BMV), :] = _bdot(v1b, w_ref[9]).astype(bf16)

    @pl.when((m >= p3s) & (m < p4s))
    def _():
        i = m - p3s
        ab = uv_ref[...].astype(bf16)
        z1 = _leaky(_bdot(ab, spu_ref[...]) + b_ref[2])
        u0b = u0_ref[pl.ds(i * BMU, BMU), :].astype(bf16)
        u1 = jnp.maximum(
            _bdot(z1.astype(bf16), w_ref[4]) + _bdot(u0b, w_ref[5])
            + b_ref[4], 0.0)
        u1_ref[...] = u1
        u1b = u1.astype(bf16)
        su1_ref[pl.ds(i * BMU, BMU), :] = u1b
        syu_ref[pl.ds(i * BMU, BMU), :] = _bdot(u1b, w_ref[8]).astype(bf16)
        z2 = _leaky(_bdot(ab, syv_ref[...]) + b_ref[7])
        spv_ref[pl.ds(i * BMU, BMU), :] = _bdot(
            z2.astype(bf16), w_ref[11]).astype(bf16)

    @pl.when((m >= p4s) & (m < p5s))
    def _():
        i = m - p4s
        ab = vu_ref[...].astype(bf16)
        z1 = _leaky(_bdot(ab, syu_ref[...]) + b_ref[6])
        spu_ref[pl.ds(i * BMV, BMV), :] = _bdot(
            z1.astype(bf16), w_ref[10]).astype(bf16)
        z2 = _leaky(_bdot(ab, spv_ref[...]) + b_ref[9])
        v1b = sv1_ref[pl.ds(i * BMV, BMV), :]
        v2_ref[...] = (_bdot(z2.astype(bf16), w_ref[14])
                       + _bdot(v1b, w_ref[15]) + b_ref[11])

    @pl.when(m >= p5s)
    def _():
        i = m - p5s
        ab = uv_ref[...].astype(bf16)
        z = _leaky(_bdot(ab, spu_ref[...]) + b_ref[8])
        u1b = su1_ref[pl.ds(i * BMU, BMU), :]
        u2_ref[...] = (_bdot(z.astype(bf16), w_ref[12])
                       + _bdot(u1b, w_ref[13]) + b_ref[10])


def _vbge(UV, VU, u0, v0, dW, db, uW, ub, lW, lb, luW, lub):
    import functools
    bf16 = jnp.bfloat16
    NU, NI = UV.shape
    nu = NU // BMU
    nv = NI // BMV
    p1s = 1
    p2s = p1s + nu
    p3s = p2s + nv
    p4s = p3s + nu
    p5s = p4s + nv
    steps = p5s + nu

    ws = jnp.stack([
        dW[0], dW[1], dW[2], dW[3],
        uW[0][:F], uW[0][F:], uW[1][:F], uW[1][F:],
        lW[0], lW[1], lW[2], lW[4],
        luW[0][:F], luW[0][F:], luW[2][:F], luW[2][F:],
    ]).astype(bf16)
    bs = jnp.stack([
        db[0], db[1], db[2], db[3], ub[0], ub[1],
        lb[0], lb[1], lb[2], lb[4], lub[0], lub[2],
    ])

    def clip(x, n):
        return jnp.clip(x, 0, n - 1)

    def uv_idx(m):
        return (jnp.where(m < p3s, clip(m - p1s, nu),
                          jnp.where(m < p5s, clip(m - p3s, nu),
                                    clip(m - p5s, nu))), 0)

    def vu_idx(m):
        return (jnp.where(m < p4s, clip(m - p2s, nv),
                          clip(m - p4s, nv)), 0)

    kfn = functools.partial(_mega_kernel, nu=nu, nv=nv)
    u1, u2, v1, v2 = pl.pallas_call(
        kfn,
        grid=(steps,),
        in_specs=[
            pl.BlockSpec((BMU, NI), uv_idx),
            pl.BlockSpec((BMV, NU), vu_idx),
            pl.BlockSpec((NU, F), lambda m: (0, 0)),
            pl.BlockSpec((NI, F), lambda m: (0, 0)),
            pl.BlockSpec((16, F, F), lambda m: (0, 0, 0)),
            pl.BlockSpec((12, F), lambda m: (0, 0)),
        ],
        out_specs=[
            pl.BlockSpec((BMU, F), lambda m: (clip(m - p3s, nu), 0)),
            pl.BlockSpec((BMU, F), lambda m: (clip(m - p5s, nu), 0)),
            pl.BlockSpec((BMV, F), lambda m: (clip(m - p2s, nv), 0)),
            pl.BlockSpec((BMV, F), lambda m: (clip(m - p4s, nv), 0)),
        ],
        out_shape=[
            jax.ShapeDtypeStruct((NU, F), jnp.float32),
            jax.ShapeDtypeStruct((NU, F), jnp.float32),
            jax.ShapeDtypeStruct((NI, F), jnp.float32),
            jax.ShapeDtypeStruct((NI, F), jnp.float32),
        ],
        scratch_shapes=[
            pltpu.VMEM((NU, F), bf16),
            pltpu.VMEM((NI, F), bf16),
            pltpu.VMEM((NU, F), bf16),
            pltpu.VMEM((NI, F), bf16),
            pltpu.VMEM((NU, F), bf16),
            pltpu.VMEM((NI, F), bf16),
        ],
        compiler_params=pltpu.CompilerParams(
            dimension_semantics=("arbitrary",)
        ),
    )(UV, VU, u0, v0, ws, bs)
    return (jnp.concatenate([u1, u2], 1), jnp.concatenate([v1, v2], 1))


def kernel(source_UV, source_VU, target_UV, target_VU, src_user_emb,
           src_item_emb, tgt_user_emb, tgt_item_emb, src_dgcn_W, src_dgcn_b,
           src_union_W, src_union_b, src_last_W, src_last_b, src_last_union_W,
           src_last_union_b, tgt_dgcn_W, tgt_dgcn_b, tgt_union_W, tgt_union_b,
           tgt_last_W, tgt_last_b, tgt_last_union_W, tgt_last_union_b):
    s_user, s_item = _vbge(source_UV, source_VU, src_user_emb, src_item_emb,
                           src_dgcn_W, src_dgcn_b, src_union_W, src_union_b,
                           src_last_W, src_last_b, src_last_union_W,
                           src_last_union_b)
    t_user, t_item = _vbge(target_UV, target_VU, tgt_user_emb, tgt_item_emb,
                           tgt_dgcn_W, tgt_dgcn_b, tgt_union_W, tgt_union_b,
                           tgt_last_W, tgt_last_b, tgt_last_union_W,
                           tgt_last_union_b)

    src_concat = jnp.concatenate([t_user[:SHARED], s_user[SHARED:]], 0)
    tgt_concat = jnp.concatenate([s_user[:SHARED], t_user[SHARED:]], 0)
    return src_concat, s_item, tgt_concat, t_item


# speedup vs baseline: 1.8200x; 1.0891x over previous
import functools

import jax
import jax.numpy as jnp
from jax.experimental import pallas as pl
from jax.experimental.pallas import tpu as pltpu

F = 128
SHARED = 2048
ALPHA = 0.1
BMU = 256
BMV = 256
F8 = jnp.float8_e4m3fn


def _leaky(x):
    return jnp.where(x > 0.0, x, ALPHA * x)


def _bdot(a, b):
    return jnp.dot(a, b, preferred_element_type=jnp.float32)


def _mega_kernel(uv_ref, vu_ref, u0_ref, v0_ref, w_ref, b_ref,
                 uv8i_ref, vu8i_ref,
                 u1_ref, u2_ref, v1_ref, v2_ref, uv8o_ref, vu8o_ref,
                 syu_ref, syv_ref, spv_ref, spu_ref, su1_ref, sv1_ref,
                 *, nu, nv):
    bf16 = jnp.bfloat16
    m = pl.program_id(0)
    p1s = 1
    p2s = p1s + nu
    p3s = p2s + nv
    p4s = p3s + nu
    p5s = p4s + nv

    @pl.when(m == 0)
    def _():
        syu_ref[...] = _bdot(u0_ref[...].astype(bf16),
                             w_ref[0]).astype(bf16)
        syv_ref[...] = _bdot(v0_ref[...].astype(bf16),
                             w_ref[1]).astype(bf16)

    @pl.when((m >= p1s) & (m < p2s))
    def _():
        i = m - p1s
        a32 = uv_ref[...]
        uv8o_ref[...] = a32.astype(F8)
        ab = a32.astype(bf16)
        z = _leaky(_bdot(ab, syv_ref[...]) + b_ref[1])
        spv_ref[pl.ds(i * BMU, BMU), :] = _bdot(
            z.astype(bf16), w_ref[3]).astype(bf16)

    @pl.when((m >= p2s) & (m < p3s))
    def _():
        i = m - p2s
        a32 = vu_ref[...]
        vu8o_ref[...] = a32.astype(F8)
        ab = a32.astype(bf16)
        z1 = _leaky(_bdot(ab, syu_ref[...]) + b_ref[0])
        spu_ref[pl.ds(i * BMV, BMV), :] = _bdot(
            z1.astype(bf16), w_ref[2]).astype(bf16)
        z2 = _leaky(_bdot(ab, spv_ref[...]) + b_ref[3])
        v0b = v0_ref[pl.ds(i * BMV, BMV), :].astype(bf16)
        v1 = jnp.maximum(
            _bdot(z2.astype(bf16), w_ref[6]) + _bdot(v0b, w_ref[7])
            + b_ref[5], 0.0)
        v1_ref[...] = v1
        v1b = v1.astype(bf16)
        sv1_ref[pl.ds(i * BMV, BMV), :] = v1b
        syv_ref[pl.ds(i * BMV, BMV), :] = _bdot(v1b, w_ref[9]).astype(bf16)

    @pl.when((m >= p3s) & (m < p4s))
    def _():
        i = m - p3s
        ab = uv8i_ref[...].astype(bf16)
        z1 = _leaky(_bdot(ab, spu_ref[...]) + b_ref[2])
        u0b = u0_ref[pl.ds(i * BMU, BMU), :].astype(bf16)
        u1 = jnp.maximum(
            _bdot(z1.astype(bf16), w_ref[4]) + _bdot(u0b, w_ref[5])
            + b_ref[4], 0.0)
        u1_ref[...] = u1
        u1b = u1.astype(bf16)
        su1_ref[pl.ds(i * BMU, BMU), :] = u1b
        syu_ref[pl.ds(i * BMU, BMU), :] = _bdot(u1b, w_ref[8]).astype(bf16)
        z2 = _leaky(_bdot(ab, syv_ref[...]) + b_ref[7])
        spv_ref[pl.ds(i * BMU, BMU), :] = _bdot(
            z2.astype(bf16), w_ref[11]).astype(bf16)

    @pl.when((m >= p4s) & (m < p5s))
    def _():
        i = m - p4s
        ab = vu8i_ref[...].astype(bf16)
        z1 = _leaky(_bdot(ab, syu_ref[...]) + b_ref[6])
        spu_ref[pl.ds(i * BMV, BMV), :] = _bdot(
            z1.astype(bf16), w_ref[10]).astype(bf16)
        z2 = _leaky(_bdot(ab, spv_ref[...]) + b_ref[9])
        v1b = sv1_ref[pl.ds(i * BMV, BMV), :]
        v2_ref[...] = (_bdot(z2.astype(bf16), w_ref[14])
                       + _bdot(v1b, w_ref[15]) + b_ref[11])

    @pl.when(m >= p5s)
    def _():
        i = m - p5s
        ab = uv8i_ref[...].astype(bf16)
        z = _leaky(_bdot(ab, spu_ref[...]) + b_ref[8])
        u1b = su1_ref[pl.ds(i * BMU, BMU), :]
        u2_ref[...] = (_bdot(z.astype(bf16), w_ref[12])
                       + _bdot(u1b, w_ref[13]) + b_ref[10])


def _vbge(UV, VU, u0, v0, dW, db, uW, ub, lW, lb, luW, lub, uv8, vu8):
    bf16 = jnp.bfloat16
    NU, NI = UV.shape
    nu = NU // BMU
    nv = NI // BMV
    p1s = 1
    p2s = p1s + nu
    p3s = p2s + nv
    p4s = p3s + nu
    p5s = p4s + nv
    steps = p5s + nu

    ws = jnp.stack([
        dW[0], dW[1], dW[2], dW[3],
        uW[0][:F], uW[0][F:], uW[1][:F], uW[1][F:],
        lW[0], lW[1], lW[2], lW[4],
        luW[0][:F], luW[0][F:], luW[2][:F], luW[2][F:],
    ]).astype(bf16)
    bs = jnp.stack([
        db[0], db[1], db[2], db[3], ub[0], ub[1],
        lb[0], lb[1], lb[2], lb[4], lub[0], lub[2],
    ])

    def clip(x, n):
        return jnp.clip(x, 0, n - 1)

    def uv8i_idx(m):
        return (jnp.where(m < p3s, nu - 1,
                          jnp.where(m < p4s, clip(m - p3s, nu),
                                    jnp.where(m < p5s, nu - 1,
                                              clip(m - p5s, nu)))), 0)

    def vu8i_idx(m):
        return (jnp.where(m < p4s, nv - 1, clip(m - p4s, nv)), 0)

    def uv8o_idx(m):
        return (jnp.where(m < p2s, clip(m - p1s, nu), 0), 0)

    def vu8o_idx(m):
        return (jnp.where(m < p3s, clip(m - p2s, nv), 0), 0)

    kfn = functools.partial(_mega_kernel, nu=nu, nv=nv)
    u1, u2, v1, v2, uv8o, vu8o = pl.pallas_call(
        kfn,
        grid=(steps,),
        in_specs=[
            pl.BlockSpec((BMU, NI), lambda m: (clip(m - p1s, nu), 0)),
            pl.BlockSpec((BMV, NU), lambda m: (clip(m - p2s, nv), 0)),
            pl.BlockSpec((NU, F), lambda m: (0, 0)),
            pl.BlockSpec((NI, F), lambda m: (0, 0)),
            pl.BlockSpec((16, F, F), lambda m: (0, 0, 0)),
            pl.BlockSpec((12, F), lambda m: (0, 0)),
            pl.BlockSpec((BMU, NI), uv8i_idx),
            pl.BlockSpec((BMV, NU), vu8i_idx),
        ],
        out_specs=[
            pl.BlockSpec((BMU, F), lambda m: (clip(m - p3s, nu), 0)),
            pl.BlockSpec((BMU, F), lambda m: (clip(m - p5s, nu), 0)),
            pl.BlockSpec((BMV, F), lambda m: (clip(m - p2s, nv), 0)),
            pl.BlockSpec((BMV, F), lambda m: (clip(m - p4s, nv), 0)),
            pl.BlockSpec((BMU, NI), uv8o_idx),
            pl.BlockSpec((BMV, NU), vu8o_idx),
        ],
        out_shape=[
            jax.ShapeDtypeStruct((NU, F), jnp.float32),
            jax.ShapeDtypeStruct((NU, F), jnp.float32),
            jax.ShapeDtypeStruct((NI, F), jnp.float32),
            jax.ShapeDtypeStruct((NI, F), jnp.float32),
            jax.ShapeDtypeStruct((NU, NI), F8),
            jax.ShapeDtypeStruct((NI, NU), F8),
        ],
        scratch_shapes=[
            pltpu.VMEM((NU, F), bf16),
            pltpu.VMEM((NI, F), bf16),
            pltpu.VMEM((NU, F), bf16),
            pltpu.VMEM((NI, F), bf16),
            pltpu.VMEM((NU, F), bf16),
            pltpu.VMEM((NI, F), bf16),
        ],
        input_output_aliases={6: 4, 7: 5},
        compiler_params=pltpu.CompilerParams(
            dimension_semantics=("arbitrary",)
        ),
    )(UV, VU, u0, v0, ws, bs, uv8, vu8)
    return (jnp.concatenate([u1, u2], 1), jnp.concatenate([v1, v2], 1),
            uv8o, vu8o)


def kernel(source_UV, source_VU, target_UV, target_VU, src_user_emb,
           src_item_emb, tgt_user_emb, tgt_item_emb, src_dgcn_W, src_dgcn_b,
           src_union_W, src_union_b, src_last_W, src_last_b, src_last_union_W,
           src_last_union_b, tgt_dgcn_W, tgt_dgcn_b, tgt_union_W, tgt_union_b,
           tgt_last_W, tgt_last_b, tgt_last_union_W, tgt_last_union_b):
    NU, NI = source_UV.shape
    uv8 = jnp.zeros((NU, NI), F8)
    vu8 = jnp.zeros((NI, NU), F8)
    s_user, s_item, uv8, vu8 = _vbge(
        source_UV, source_VU, src_user_emb, src_item_emb,
        src_dgcn_W, src_dgcn_b, src_union_W, src_union_b,
        src_last_W, src_last_b, src_last_union_W, src_last_union_b,
        uv8, vu8)
    t_user, t_item, uv8, vu8 = _vbge(
        target_UV, target_VU, tgt_user_emb, tgt_item_emb,
        tgt_dgcn_W, tgt_dgcn_b, tgt_union_W, tgt_union_b,
        tgt_last_W, tgt_last_b, tgt_last_union_W, tgt_last_union_b,
        uv8, vu8)

    src_concat = jnp.concatenate([t_user[:SHARED], s_user[SHARED:]], 0)
    tgt_concat = jnp.concatenate([s_user[:SHARED], t_user[SHARED:]], 0)
    return src_concat, s_item, tgt_concat, t_item
